# baseline (speedup 1.0000x reference)
import jax
import jax.numpy as jnp
from jax import lax
from jax.experimental import pallas as pl
from jax.experimental.pallas import tpu as pltpu

_B = 8
_S = 197
_D = 768
_C = 512
_K = 16
_NCLS = 1000
_NEG = float("-inf")


def _dot_qk(q2d, cw_ref):
    return lax.dot_general(q2d, cw_ref[...], (((1,), (1,)), ((), ())),
                           preferred_element_type=jnp.float32)


def _body(q_ref, qn_ref, cw_ref, clsw_ref, clsb_ref, y_ref, h_ref, qk_ref):
    b = pl.program_id(0)
    par = lax.rem(b, jnp.int32(2))
    qb = q_ref[0]

    @pl.when(b == 0)
    def _prologue():
        qk_ref[pl.ds(0, 1)] = _dot_qk(q_ref[0], cw_ref)[None]

    @pl.when(b < _B - 1)
    def _ahead():
        qk_ref[pl.ds(1 - par, 1)] = _dot_qk(qn_ref[0], cw_ref)[None]

    qk = qk_ref[pl.ds(par, 1)][0]

    iota_s = lax.broadcasted_iota(jnp.int32, (_S, _C), 0)
    qk0 = jnp.where(iota_s == 0, _NEG, qk)

    w = qk0
    m0 = jnp.max(qk0, axis=0, keepdims=True)
    for _ in range(_K):
        pos = jnp.argmax(w, axis=0)
        w = jnp.where(iota_s == pos[None, :], _NEG, w)

    e = jnp.exp(qk0 - m0)
    a = jnp.where(w == _NEG, e, 0.0)
    denom = jnp.sum(a, axis=0, keepdims=True)
    wb = jnp.sum(a / denom, axis=1, keepdims=True) * (1.0 / _C)
    h = jnp.sum(wb * qb, axis=0, keepdims=True)
    h = jnp.maximum(h, 0.0)

    iota_b = lax.broadcasted_iota(jnp.int32, (_B, _D), 0)
    hb = jnp.where(iota_b == b, h, 0.0)
    h_ref[...] = jnp.where(b == 0, hb, h_ref[...] + hb)

    @pl.when(b == _B - 1)
    def _classifier():
        y = lax.dot_general(h_ref[...], clsw_ref[...], (((1,), (1,)), ((), ())),
                            preferred_element_type=jnp.float32)
        y_ref[...] = y + clsb_ref[...]


def kernel(q_full, concept_w, cls_w, cls_b):
    clsb = cls_b.reshape(1, _NCLS)
    return pl.pallas_call(
        _body,
        grid=(_B,),
        in_specs=[
            pl.BlockSpec((1, _S, _D), lambda b: (b, 0, 0)),
            pl.BlockSpec((1, _S, _D), lambda b: (jnp.minimum(b + 1, _B - 1), 0, 0)),
            pl.BlockSpec((_C, _D), lambda b: (0, 0)),
            pl.BlockSpec((_NCLS, _D), lambda b: (0, 0)),
            pl.BlockSpec((1, _NCLS), lambda b: (0, 0)),
        ],
        out_specs=pl.BlockSpec((_B, _NCLS), lambda b: (0, 0)),
        out_shape=jax.ShapeDtypeStruct((_B, _NCLS), jnp.float32),
        scratch_shapes=[pltpu.VMEM((_B, _D), jnp.float32),
                        pltpu.VMEM((2, _S, _C), jnp.float32)],
    )(q_full, q_full, concept_w, cls_w, clsb)


# speedup vs baseline: 1.0917x; 1.0917x over previous
import jax
import jax.numpy as jnp
from jax import lax
from jax.experimental import pallas as pl
from jax.experimental.pallas import tpu as pltpu

_B = 8
_S = 197
_D = 768
_C = 512
_K = 16
_NCLS = 1000
_NEG = float("-inf")


def _body(q_ref, cw_ref, clsw_hbm, clsb_ref, y_ref, h_ref, clsw_vmem, sem):
    b = pl.program_id(0)

    @pl.when(b == 0)
    def _start_clsw():
        pltpu.make_async_copy(clsw_hbm, clsw_vmem, sem).start()

    qb = q_ref[0]
    qk = lax.dot_general(qb, cw_ref[...], (((1,), (1,)), ((), ())),
                         preferred_element_type=jnp.float32)

    iota_s = lax.broadcasted_iota(jnp.int32, (_S, _C), 0)
    qk0 = jnp.where(iota_s == 0, _NEG, qk)

    w = qk0
    m0 = jnp.max(qk0, axis=0, keepdims=True)
    for _ in range(_K):
        pos = jnp.argmax(w, axis=0)
        w = jnp.where(iota_s == pos[None, :], _NEG, w)

    e = jnp.exp(qk0 - m0)
    a = jnp.where(w == _NEG, e, 0.0)
    denom = jnp.sum(a, axis=0, keepdims=True)
    wb = jnp.sum(a / denom, axis=1, keepdims=True) * (1.0 / _C)
    h = jnp.sum(wb * qb, axis=0, keepdims=True)
    h = jnp.maximum(h, 0.0)

    iota_b = lax.broadcasted_iota(jnp.int32, (_B, _D), 0)
    hb = jnp.where(iota_b == b, h, 0.0)
    h_ref[...] = jnp.where(b == 0, hb, h_ref[...] + hb)

    @pl.when(b == _B - 1)
    def _classifier():
        pltpu.make_async_copy(clsw_hbm, clsw_vmem, sem).wait()
        y = lax.dot_general(h_ref[...], clsw_vmem[...], (((1,), (1,)), ((), ())),
                            preferred_element_type=jnp.float32)
        y_ref[...] = y + clsb_ref[...]


def kernel(q_full, concept_w, cls_w, cls_b):
    clsb = cls_b.reshape(1, _NCLS)
    return pl.pallas_call(
        _body,
        grid=(_B,),
        in_specs=[
            pl.BlockSpec((1, _S, _D), lambda b: (b, 0, 0)),
            pl.BlockSpec((_C, _D), lambda b: (0, 0)),
            pl.BlockSpec(memory_space=pl.ANY),
            pl.BlockSpec((1, _NCLS), lambda b: (0, 0)),
        ],
        out_specs=pl.BlockSpec((_B, _NCLS), lambda b: (0, 0)),
        out_shape=jax.ShapeDtypeStruct((_B, _NCLS), jnp.float32),
        scratch_shapes=[pltpu.VMEM((_B, _D), jnp.float32),
                        pltpu.VMEM((_NCLS, _D), jnp.float32),
                        pltpu.SemaphoreType.DMA],
    )(q_full, concept_w, cls_w, clsb)
